# baseline (speedup 1.0000x reference)
import functools

import jax
import jax.numpy as jnp
from jax import lax
from jax.experimental import pallas as pl
from jax.experimental.pallas import tpu as pltpu
from jax.experimental.pallas import tpu_sc as plsc

_N = 10000
_D = 128
_H = 16
_C = 40

_NP = 10112
_PR = _NP // 8
_B7 = _N - 7 * _PR
_NSUB = 16
_NC = 2
_RPT = _NP // _NSUB
_CHUNK = 128
_NB = 4

_SELU_ALPHA = 1.6732632423543772
_SELU_SCALE = 1.0507009873554805


def _xform_chunk(idx_v, j):
    for i in range(8):
        v = idx_v[j, pl.ds(16 * i, 16)]
        q = ((v >> 4) * 830) >> 16
        idx_v[j, pl.ds(16 * i, 16)] = (v << 3) - q * 10111


def _schedule(ec, nc):
    nw = nc * _NSUB
    n_lo, extra = divmod(ec, nw)
    max_n = n_lo + (1 if extra else 0)
    n_batches = -(-max_n // _NB)
    assert n_lo >= 2 * _NB and n_batches >= 3
    return n_lo, extra, max_n, n_batches


def _sc_degree(ec, nc):
    n_lo, extra, max_n, nbat = _schedule(ec, nc)
    mesh = plsc.VectorSubcoreMesh(
        core_axis_name="c", subcore_axis_name="s", num_cores=nc)

    @functools.partial(
        pl.kernel,
        out_type=jax.ShapeDtypeStruct((nc, _NP, _H), jnp.float32),
        mesh=mesh,
        scratch_types=[
            pltpu.VMEM((max_n, _CHUNK), jnp.int32),
            pltpu.VMEM((_CHUNK, _H), jnp.float32),
            pltpu.VMEM_SHARED((_NP, _H), jnp.float32),
            [pltpu.SemaphoreType.DMA] * _NB,
        ],
        compiler_params=pltpu.CompilerParams(use_tc_tiling_on_sc=False),
    )
    def deg_kernel(dst_hbm, ones_hbm, zeros_hbm, out_hbm, dst_v, ones_v, acc,
                   sems):
        c = lax.axis_index("c")
        s = lax.axis_index("s")
        wid = s * nc + c
        n = n_lo + jnp.where(wid < extra, 1, 0)
        start = wid * n_lo + jnp.minimum(wid, extra)
        pltpu.sync_copy(zeros_hbm.at[pl.ds(s * _RPT, _RPT)],
                        acc.at[pl.ds(s * _RPT, _RPT)])
        pltpu.sync_copy(ones_hbm, ones_v)
        if extra:
            @pl.when(n == max_n)
            def _():
                pltpu.sync_copy(dst_hbm.at[pl.ds(start, max_n)], dst_v)

            @pl.when(n == n_lo)
            def _():
                pltpu.sync_copy(dst_hbm.at[pl.ds(start, n_lo)],
                                dst_v.at[pl.ds(0, n_lo)])
        else:
            pltpu.sync_copy(dst_hbm.at[pl.ds(start, n_lo)], dst_v)
        for j in range(2 * _NB):
            _xform_chunk(dst_v, j)
        plsc.subcore_barrier()

        for b in range(_NB):
            pltpu.async_copy(ones_v, acc.at[dst_v.at[b]], sems[b], add=True)

        @pl.loop(0, nbat - 2)
        def _(k):
            for b in range(_NB):
                j2 = (k + 2) * _NB + b
                @pl.when(j2 < n)
                def _():
                    _xform_chunk(dst_v, j2)
            for b in range(_NB):
                j = k * _NB + b
                pltpu.make_async_copy(ones_v, acc.at[dst_v.at[j]],
                                      sems[b]).wait()
                pltpu.async_copy(ones_v, acc.at[dst_v.at[j + _NB]], sems[b],
                                 add=True)

        for b in range(_NB):
            j = (nbat - 2) * _NB + b
            pltpu.make_async_copy(ones_v, acc.at[dst_v.at[j]], sems[b]).wait()
            j2 = (nbat - 1) * _NB + b
            @pl.when(j2 < n)
            def _():
                pltpu.async_copy(ones_v, acc.at[dst_v.at[j2]], sems[b],
                                 add=True)
        for b in range(_NB):
            j2 = (nbat - 1) * _NB + b
            @pl.when(j2 < n)
            def _():
                pltpu.make_async_copy(ones_v, acc.at[dst_v.at[j2]],
                                      sems[b]).wait()

        plsc.subcore_barrier()
        pltpu.sync_copy(acc.at[pl.ds(s * _RPT, _RPT)],
                        out_hbm.at[c, pl.ds(s * _RPT, _RPT)])

    return deg_kernel


def _sc_prop(ec, nc):
    n_lo, extra, max_n, nbat = _schedule(ec, nc)
    mesh = plsc.VectorSubcoreMesh(
        core_axis_name="c", subcore_axis_name="s", num_cores=nc)

    @functools.partial(
        pl.kernel,
        out_type=jax.ShapeDtypeStruct((nc, _NP, _H), jnp.float32),
        mesh=mesh,
        scratch_types=[
            pltpu.VMEM((max_n, _CHUNK), jnp.int32),
            pltpu.VMEM((max_n, _CHUNK), jnp.int32),
            [pltpu.VMEM((_CHUNK, _H), jnp.float32)] * _NB,
            [pltpu.SemaphoreType.DMA] * _NB,
            [pltpu.SemaphoreType.DMA] * _NB,
            pltpu.VMEM_SHARED((_NP, _H), jnp.float32),
            pltpu.VMEM_SHARED((_NP, _H), jnp.float32),
        ],
        compiler_params=pltpu.CompilerParams(use_tc_tiling_on_sc=False),
    )
    def prop_kernel(u_hbm, src_hbm, dst_hbm, zeros_hbm, out_hbm,
                    src_v, dst_v, bufs, gsems, ssems, acc, utab):
        c = lax.axis_index("c")
        s = lax.axis_index("s")
        wid = s * nc + c
        n = n_lo + jnp.where(wid < extra, 1, 0)
        start = wid * n_lo + jnp.minimum(wid, extra)
        pltpu.sync_copy(zeros_hbm.at[pl.ds(s * _RPT, _RPT)],
                        acc.at[pl.ds(s * _RPT, _RPT)])
        pltpu.sync_copy(u_hbm.at[pl.ds(s * _RPT, _RPT)],
                        utab.at[pl.ds(s * _RPT, _RPT)])
        if extra:
            @pl.when(n == max_n)
            def _():
                pltpu.sync_copy(src_hbm.at[pl.ds(start, max_n)], src_v)
                pltpu.sync_copy(dst_hbm.at[pl.ds(start, max_n)], dst_v)

            @pl.when(n == n_lo)
            def _():
                pltpu.sync_copy(src_hbm.at[pl.ds(start, n_lo)],
                                src_v.at[pl.ds(0, n_lo)])
                pltpu.sync_copy(dst_hbm.at[pl.ds(start, n_lo)],
                                dst_v.at[pl.ds(0, n_lo)])
        else:
            pltpu.sync_copy(src_hbm.at[pl.ds(start, n_lo)], src_v)
            pltpu.sync_copy(dst_hbm.at[pl.ds(start, n_lo)], dst_v)
        for j in range(2 * _NB):
            _xform_chunk(src_v, j)
            _xform_chunk(dst_v, j)
        plsc.subcore_barrier()

        for b in range(_NB):
            pltpu.async_copy(utab.at[src_v.at[b]], bufs[b], gsems[b])

        @pl.loop(0, nbat - 2)
        def _(k):
            for b in range(_NB):
                j2 = (k + 2) * _NB + b
                @pl.when(j2 < n)
                def _():
                    _xform_chunk(src_v, j2)
                    _xform_chunk(dst_v, j2)
            for b in range(_NB):
                j = k * _NB + b
                pltpu.make_async_copy(utab.at[src_v.at[j]], bufs[b],
                                      gsems[b]).wait()
                pltpu.async_copy(bufs[b], acc.at[dst_v.at[j]], ssems[b],
                                 add=True)
            for b in range(_NB):
                j = k * _NB + b
                pltpu.make_async_copy(bufs[b], acc.at[dst_v.at[j]],
                                      ssems[b]).wait()
                pltpu.async_copy(utab.at[src_v.at[j + _NB]], bufs[b],
                                 gsems[b])

        for b in range(_NB):
            j = (nbat - 2) * _NB + b
            pltpu.make_async_copy(utab.at[src_v.at[j]], bufs[b],
                                  gsems[b]).wait()
            pltpu.async_copy(bufs[b], acc.at[dst_v.at[j]], ssems[b], add=True)
        for b in range(_NB):
            j = (nbat - 2) * _NB + b
            pltpu.make_async_copy(bufs[b], acc.at[dst_v.at[j]],
                                  ssems[b]).wait()
            j2 = (nbat - 1) * _NB + b
            @pl.when(j2 < n)
            def _():
                pltpu.async_copy(utab.at[src_v.at[j2]], bufs[b], gsems[b])
        for b in range(_NB):
            j2 = (nbat - 1) * _NB + b
            @pl.when(j2 < n)
            def _():
                pltpu.make_async_copy(utab.at[src_v.at[j2]], bufs[b],
                                      gsems[b]).wait()
                pltpu.async_copy(bufs[b], acc.at[dst_v.at[j2]], ssems[b],
                                 add=True)
        for b in range(_NB):
            j2 = (nbat - 1) * _NB + b
            @pl.when(j2 < n)
            def _():
                pltpu.make_async_copy(bufs[b], acc.at[dst_v.at[j2]],
                                      ssems[b]).wait()

        plsc.subcore_barrier()
        pltpu.sync_copy(acc.at[pl.ds(s * _RPT, _RPT)],
                        out_hbm.at[c, pl.ds(s * _RPT, _RPT)])

    return prop_kernel


def _pmask(val, other=0.0):
    rows = lax.broadcasted_iota(jnp.int32, (_PR, _D), 0)
    blocks = lax.broadcasted_iota(jnp.int32, (_PR, _D), 1) // _H
    return jnp.where((blocks < 7) | (rows < _B7), val, other)


def _mm_body(x_ref, w1_ref, h0_ref):
    w1 = w1_ref[...]
    parts = []
    for b in range(7):
        parts.append(jnp.dot(x_ref[pl.ds(b * _PR, _PR), :], w1,
                             preferred_element_type=jnp.float32))
    t7 = jnp.dot(x_ref[pl.ds(7 * _PR, _B7), :], w1,
                 preferred_element_type=jnp.float32)
    parts.append(jnp.concatenate(
        [t7, jnp.zeros((_PR - _B7, _H), jnp.float32)], axis=0))
    h0_ref[...] = jnp.concatenate(parts, axis=1)


def _prep_body(h0_ref, degp_ref, dinv_ref, u0_ref):
    deg = jnp.sum(degp_ref[...], axis=0) + 1.0
    dinv = lax.rsqrt(deg)
    dinv_ref[...] = dinv
    u0_ref[...] = _pmask(dinv * h0_ref[...])


def _mid_body(p_ref, u_ref, dinv_ref, o_ref):
    t = jnp.sum(p_ref[...], axis=0) + u_ref[...]
    d = dinv_ref[...]
    o_ref[...] = _pmask(d * d * t)


def _bn_body(p_ref, u_ref, dinv_ref, b1_ref, gamma_ref, beta_ref, o_ref):
    t = jnp.sum(p_ref[...], axis=0) + u_ref[...]
    d = dinv_ref[...]
    h = d * t + b1_ref[...]
    ii = lax.broadcasted_iota(jnp.int32, (_D, _D), 0) % _H
    jj = lax.broadcasted_iota(jnp.int32, (_D, _D), 1) % _H
    fold = (ii == jj).astype(jnp.float32)
    hm = _pmask(h)
    mean = jnp.dot(jnp.sum(hm, axis=0, keepdims=True), fold,
                   preferred_element_type=jnp.float32) * (1.0 / _N)
    dev = _pmask(h - mean)
    var = jnp.dot(jnp.sum(dev * dev, axis=0, keepdims=True), fold,
                  preferred_element_type=jnp.float32) * (1.0 / _N)
    hn = (h - mean) * lax.rsqrt(var + 1e-5) * gamma_ref[...] + beta_ref[...]
    sel = _SELU_SCALE * jnp.where(hn > 0, hn, _SELU_ALPHA * (jnp.exp(hn) - 1.0))
    o_ref[...] = _pmask(d * sel)


def _fin_body(p_ref, u_ref, dinv_ref, w2_ref, b2_ref, o_ref):
    t = jnp.sum(p_ref[...], axis=0) + u_ref[...]
    g = dinv_ref[...] * t
    w2 = w2_ref[...]
    b2t = b2_ref[...].reshape(_C, 1)
    for b in range(8):
        rows = _PR if b < 7 else _B7
        gb = lax.slice(g, (0, b * _H), (rows, (b + 1) * _H))
        zt = lax.dot_general(w2, gb, (((0,), (1,)), ((), ())),
                             preferred_element_type=jnp.float32) + b2t
        m = jnp.max(zt, axis=0, keepdims=True)
        e = jnp.exp(zt - m)
        lsm = (zt - m) - jnp.log(jnp.sum(e, axis=0, keepdims=True))
        o_ref[:, pl.ds(b * _PR, rows)] = lsm


def _sds(shape):
    return jax.ShapeDtypeStruct(shape, jnp.float32)


def kernel(x, edge_index, W1, b1, gamma, beta, W2, b2):
    e = edge_index.shape[1]
    dst = edge_index[1]
    if e % _CHUNK:
        pad = _CHUNK - e % _CHUNK
        fill = jnp.full((pad,), _N, jnp.int32)
        dst = jnp.concatenate([dst, fill])
    ec = dst.shape[0] // _CHUNK
    dst2 = dst.reshape(ec, _CHUNK)
    (dst2, edge_index2) = lax.optimization_barrier((dst2, edge_index))
    src = edge_index2[0]
    if e % _CHUNK:
        src = jnp.concatenate([src, jnp.full((_CHUNK - e % _CHUNK,), _N,
                                             jnp.int32)])
    src2 = src.reshape(ec, _CHUNK)
    zeros = jnp.zeros((_NP, _H), jnp.float32)
    ones = jnp.ones((_CHUNK, _H), jnp.float32)
    b1r = jnp.tile(b1, 8).reshape(1, _D)
    gammar = jnp.tile(gamma, 8).reshape(1, _D)
    betar = jnp.tile(beta, 8).reshape(1, _D)
    b2r = b2.reshape(1, _C)

    def packed(a):
        return a.reshape(a.shape[:-2] + (_PR, _D))

    def table(a):
        return a.reshape((_NP, _H))

    degp = packed(_sc_degree(ec, _NC)(dst2, ones, zeros))
    h0 = pl.pallas_call(_mm_body, out_shape=_sds((_PR, _D)))(x, W1)
    dinv, u0 = pl.pallas_call(
        _prep_body, out_shape=(_sds((_PR, _D)), _sds((_PR, _D))))(h0, degp)

    prop = _sc_prop(ec, _NC)
    p1 = packed(prop(table(u0), src2, dst2, zeros))
    u1 = pl.pallas_call(_mid_body, out_shape=_sds((_PR, _D)))(p1, u0, dinv)
    p2 = packed(prop(table(u1), src2, dst2, zeros))
    u2 = pl.pallas_call(_bn_body, out_shape=_sds((_PR, _D)))(
        p2, u1, dinv, b1r, gammar, betar)
    p3 = packed(prop(table(u2), src2, dst2, zeros))
    out_t = pl.pallas_call(_fin_body, out_shape=_sds((_C, _N)))(
        p3, u2, dinv, W2, b2r)
    return out_t.T


# speedup vs baseline: 1.0389x; 1.0389x over previous
import functools

import jax
import jax.numpy as jnp
from jax import lax
from jax.experimental import pallas as pl
from jax.experimental.pallas import tpu as pltpu
from jax.experimental.pallas import tpu_sc as plsc

_N = 10000
_D = 128
_H = 16
_C = 40

_NP = 10112
_PR = _NP // 8
_B7 = _N - 7 * _PR
_NSUB = 16
_NC = 2
_RPT = _NP // _NSUB
_CHUNK = 128
_NB = 4

_SELU_ALPHA = 1.6732632423543772
_SELU_SCALE = 1.0507009873554805


def _xform_chunk(idx_v, j):
    for i in range(8):
        v = idx_v[j, pl.ds(16 * i, 16)]
        q = ((v >> 4) * 830) >> 16
        idx_v[j, pl.ds(16 * i, 16)] = (v << 3) - q * 10111


def _schedule(ec, nc):
    nw = nc * _NSUB
    n_lo, extra = divmod(ec, nw)
    max_n = n_lo + (1 if extra else 0)
    n_batches = -(-max_n // _NB)
    assert n_lo >= 2 * _NB and n_batches >= 3
    return n_lo, extra, max_n, n_batches


def _sc_degree(ec, nc):
    n_lo, extra, max_n, nbat = _schedule(ec, nc)
    mesh = plsc.VectorSubcoreMesh(
        core_axis_name="c", subcore_axis_name="s", num_cores=nc)

    @functools.partial(
        pl.kernel,
        out_type=jax.ShapeDtypeStruct((nc, _NP, _H), jnp.float32),
        mesh=mesh,
        scratch_types=[
            pltpu.VMEM((max_n, _CHUNK), jnp.int32),
            pltpu.VMEM((_CHUNK, _H), jnp.float32),
            pltpu.VMEM_SHARED((_NP, _H), jnp.float32),
            [pltpu.SemaphoreType.DMA] * _NB,
        ],
        compiler_params=pltpu.CompilerParams(use_tc_tiling_on_sc=False),
    )
    def deg_kernel(dst_hbm, ones_hbm, zeros_hbm, out_hbm, dst_v, ones_v, acc,
                   sems):
        c = lax.axis_index("c")
        s = lax.axis_index("s")
        wid = s * nc + c
        n = n_lo + jnp.where(wid < extra, 1, 0)
        start = wid * n_lo + jnp.minimum(wid, extra)
        pltpu.sync_copy(zeros_hbm.at[pl.ds(s * _RPT, _RPT)],
                        acc.at[pl.ds(s * _RPT, _RPT)])
        pltpu.sync_copy(ones_hbm, ones_v)
        if extra:
            @pl.when(n == max_n)
            def _():
                pltpu.sync_copy(dst_hbm.at[pl.ds(start, max_n)], dst_v)

            @pl.when(n == n_lo)
            def _():
                pltpu.sync_copy(dst_hbm.at[pl.ds(start, n_lo)],
                                dst_v.at[pl.ds(0, n_lo)])
        else:
            pltpu.sync_copy(dst_hbm.at[pl.ds(start, n_lo)], dst_v)
        for j in range(2 * _NB):
            _xform_chunk(dst_v, j)
        plsc.subcore_barrier()

        for b in range(_NB):
            pltpu.async_copy(ones_v, acc.at[dst_v.at[b]], sems[b], add=True)

        @pl.loop(0, nbat - 2)
        def _(k):
            for b in range(_NB):
                j2 = (k + 2) * _NB + b
                @pl.when(j2 < n)
                def _():
                    _xform_chunk(dst_v, j2)
            for b in range(_NB):
                j = k * _NB + b
                pltpu.make_async_copy(ones_v, acc.at[dst_v.at[j]],
                                      sems[b]).wait()
                pltpu.async_copy(ones_v, acc.at[dst_v.at[j + _NB]], sems[b],
                                 add=True)

        for b in range(_NB):
            j = (nbat - 2) * _NB + b
            pltpu.make_async_copy(ones_v, acc.at[dst_v.at[j]], sems[b]).wait()
            j2 = (nbat - 1) * _NB + b
            @pl.when(j2 < n)
            def _():
                pltpu.async_copy(ones_v, acc.at[dst_v.at[j2]], sems[b],
                                 add=True)
        for b in range(_NB):
            j2 = (nbat - 1) * _NB + b
            @pl.when(j2 < n)
            def _():
                pltpu.make_async_copy(ones_v, acc.at[dst_v.at[j2]],
                                      sems[b]).wait()

        plsc.subcore_barrier()
        pltpu.sync_copy(acc.at[pl.ds(s * _RPT, _RPT)],
                        out_hbm.at[c, pl.ds(s * _RPT, _RPT)])

    return deg_kernel


def _sc_prop(ec, nc):
    n_lo, extra, max_n, nbat = _schedule(ec, nc)
    mesh = plsc.VectorSubcoreMesh(
        core_axis_name="c", subcore_axis_name="s", num_cores=nc)

    @functools.partial(
        pl.kernel,
        out_type=jax.ShapeDtypeStruct((nc, _NP, _H), jnp.float32),
        mesh=mesh,
        scratch_types=[
            pltpu.VMEM((max_n, _CHUNK), jnp.int32),
            pltpu.VMEM((max_n, _CHUNK), jnp.int32),
            [pltpu.VMEM((_CHUNK, _H), jnp.float32)] * _NB,
            [pltpu.SemaphoreType.DMA] * _NB,
            [pltpu.SemaphoreType.DMA] * _NB,
            pltpu.VMEM_SHARED((_NP, _H), jnp.float32),
            pltpu.VMEM_SHARED((_NP, _H), jnp.float32),
        ],
        compiler_params=pltpu.CompilerParams(use_tc_tiling_on_sc=False),
    )
    def prop_kernel(u_hbm, src_hbm, dst_hbm, zeros_hbm, out_hbm,
                    src_v, dst_v, bufs, gsems, ssems, acc, utab):
        c = lax.axis_index("c")
        s = lax.axis_index("s")
        wid = s * nc + c
        n = n_lo + jnp.where(wid < extra, 1, 0)
        start = wid * n_lo + jnp.minimum(wid, extra)
        pltpu.sync_copy(zeros_hbm.at[pl.ds(s * _RPT, _RPT)],
                        acc.at[pl.ds(s * _RPT, _RPT)])
        pltpu.sync_copy(u_hbm.at[pl.ds(s * _RPT, _RPT)],
                        utab.at[pl.ds(s * _RPT, _RPT)])
        if extra:
            @pl.when(n == max_n)
            def _():
                pltpu.sync_copy(src_hbm.at[pl.ds(start, max_n)], src_v)
                pltpu.sync_copy(dst_hbm.at[pl.ds(start, max_n)], dst_v)

            @pl.when(n == n_lo)
            def _():
                pltpu.sync_copy(src_hbm.at[pl.ds(start, n_lo)],
                                src_v.at[pl.ds(0, n_lo)])
                pltpu.sync_copy(dst_hbm.at[pl.ds(start, n_lo)],
                                dst_v.at[pl.ds(0, n_lo)])
        else:
            pltpu.sync_copy(src_hbm.at[pl.ds(start, n_lo)], src_v)
            pltpu.sync_copy(dst_hbm.at[pl.ds(start, n_lo)], dst_v)
        for j in range(2 * _NB):
            _xform_chunk(src_v, j)
            _xform_chunk(dst_v, j)
        plsc.subcore_barrier()

        for b in range(_NB):
            pltpu.async_copy(utab.at[src_v.at[b]], bufs[b], gsems[b])

        @pl.loop(0, nbat - 2)
        def _(k):
            for b in range(_NB):
                j2 = (k + 2) * _NB + b
                @pl.when(j2 < n)
                def _():
                    _xform_chunk(src_v, j2)
                    _xform_chunk(dst_v, j2)
            for b in range(_NB):
                j = k * _NB + b
                pltpu.make_async_copy(utab.at[src_v.at[j]], bufs[b],
                                      gsems[b]).wait()
                pltpu.async_copy(bufs[b], acc.at[dst_v.at[j]], ssems[b],
                                 add=True)
            for b in range(_NB):
                j = k * _NB + b
                pltpu.make_async_copy(bufs[b], acc.at[dst_v.at[j]],
                                      ssems[b]).wait()
                pltpu.async_copy(utab.at[src_v.at[j + _NB]], bufs[b],
                                 gsems[b])

        for b in range(_NB):
            j = (nbat - 2) * _NB + b
            pltpu.make_async_copy(utab.at[src_v.at[j]], bufs[b],
                                  gsems[b]).wait()
            pltpu.async_copy(bufs[b], acc.at[dst_v.at[j]], ssems[b], add=True)
        for b in range(_NB):
            j = (nbat - 2) * _NB + b
            pltpu.make_async_copy(bufs[b], acc.at[dst_v.at[j]],
                                  ssems[b]).wait()
            j2 = (nbat - 1) * _NB + b
            @pl.when(j2 < n)
            def _():
                pltpu.async_copy(utab.at[src_v.at[j2]], bufs[b], gsems[b])
        for b in range(_NB):
            j2 = (nbat - 1) * _NB + b
            @pl.when(j2 < n)
            def _():
                pltpu.make_async_copy(utab.at[src_v.at[j2]], bufs[b],
                                      gsems[b]).wait()
                pltpu.async_copy(bufs[b], acc.at[dst_v.at[j2]], ssems[b],
                                 add=True)
        for b in range(_NB):
            j2 = (nbat - 1) * _NB + b
            @pl.when(j2 < n)
            def _():
                pltpu.make_async_copy(bufs[b], acc.at[dst_v.at[j2]],
                                      ssems[b]).wait()

        plsc.subcore_barrier()
        pltpu.sync_copy(acc.at[pl.ds(s * _RPT, _RPT)],
                        out_hbm.at[c, pl.ds(s * _RPT, _RPT)])

    return prop_kernel


def _pmask(val, other=0.0):
    rows = lax.broadcasted_iota(jnp.int32, (_PR, _D), 0)
    blocks = lax.broadcasted_iota(jnp.int32, (_PR, _D), 1) // _H
    return jnp.where((blocks < 7) | (rows < _B7), val, other)


def _mm_body(x_ref, w1_ref, h0_ref):
    w1 = w1_ref[...]
    parts = []
    for b in range(7):
        parts.append(jnp.dot(x_ref[pl.ds(b * _PR, _PR), :], w1,
                             preferred_element_type=jnp.float32))
    t7 = jnp.dot(x_ref[pl.ds(7 * _PR, _B7), :], w1,
                 preferred_element_type=jnp.float32)
    parts.append(jnp.concatenate(
        [t7, jnp.zeros((_PR - _B7, _H), jnp.float32)], axis=0))
    h0_ref[...] = jnp.concatenate(parts, axis=1)


def _prep_body(h0_ref, degp_ref, dinv_ref, u0_ref):
    deg = jnp.sum(degp_ref[...], axis=0) + 1.0
    dinv = lax.rsqrt(deg)
    dinv_ref[...] = dinv
    u0_ref[...] = _pmask(dinv * h0_ref[...])


def _mid_body(p_ref, u_ref, dinv_ref, o_ref):
    t = jnp.sum(p_ref[...], axis=0) + u_ref[...]
    d = dinv_ref[...]
    o_ref[...] = _pmask(d * d * t)


def _bn_body(p_ref, u_ref, dinv_ref, b1_ref, gamma_ref, beta_ref, o_ref):
    t = jnp.sum(p_ref[...], axis=0) + u_ref[...]
    d = dinv_ref[...]
    h = d * t + b1_ref[...]
    ii = lax.broadcasted_iota(jnp.int32, (_D, _D), 0) % _H
    jj = lax.broadcasted_iota(jnp.int32, (_D, _D), 1) % _H
    fold = (ii == jj).astype(jnp.float32)
    hm = _pmask(h)
    mean = jnp.dot(jnp.sum(hm, axis=0, keepdims=True), fold,
                   preferred_element_type=jnp.float32) * (1.0 / _N)
    dev = _pmask(h - mean)
    var = jnp.dot(jnp.sum(dev * dev, axis=0, keepdims=True), fold,
                  preferred_element_type=jnp.float32) * (1.0 / _N)
    hn = (h - mean) * lax.rsqrt(var + 1e-5) * gamma_ref[...] + beta_ref[...]
    sel = _SELU_SCALE * jnp.where(hn > 0, hn, _SELU_ALPHA * (jnp.exp(hn) - 1.0))
    o_ref[...] = _pmask(d * sel)


def _fin_body(p_ref, u_ref, dinv_ref, w2_ref, b2_ref, o_ref):
    t = jnp.sum(p_ref[...], axis=0) + u_ref[...]
    g = dinv_ref[...] * t
    w2 = w2_ref[...]
    b2t = b2_ref[...].reshape(_C, 1)
    for b in range(8):
        rows = _PR if b < 7 else _B7
        gb = lax.slice(g, (0, b * _H), (rows, (b + 1) * _H))
        zt = lax.dot_general(w2, gb, (((0,), (1,)), ((), ())),
                             preferred_element_type=jnp.float32) + b2t
        m = jnp.max(zt, axis=0, keepdims=True)
        e = jnp.exp(zt - m)
        lsm = (zt - m) - jnp.log(jnp.sum(e, axis=0, keepdims=True))
        o_ref[:, pl.ds(b * _PR, rows)] = lsm


def _sds(shape):
    return jax.ShapeDtypeStruct(shape, jnp.float32)


def kernel(x, edge_index, W1, b1, gamma, beta, W2, b2):
    e = edge_index.shape[1]
    src, dst = edge_index[0], edge_index[1]
    if e % _CHUNK:
        pad = _CHUNK - e % _CHUNK
        fill = jnp.full((pad,), _N, jnp.int32)
        src = jnp.concatenate([src, fill])
        dst = jnp.concatenate([dst, fill])
    ec = src.shape[0] // _CHUNK
    src2 = src.reshape(ec, _CHUNK)
    dst2 = dst.reshape(ec, _CHUNK)
    zeros = jnp.zeros((_NP, _H), jnp.float32)
    ones = jnp.ones((_CHUNK, _H), jnp.float32)
    b1r = jnp.tile(b1, 8).reshape(1, _D)
    gammar = jnp.tile(gamma, 8).reshape(1, _D)
    betar = jnp.tile(beta, 8).reshape(1, _D)
    b2r = b2.reshape(1, _C)

    def packed(a):
        return a.reshape(a.shape[:-2] + (_PR, _D))

    def table(a):
        return a.reshape((_NP, _H))

    degp = packed(_sc_degree(ec, _NC)(dst2, ones, zeros))
    h0 = pl.pallas_call(_mm_body, out_shape=_sds((_PR, _D)))(x, W1)
    dinv, u0 = pl.pallas_call(
        _prep_body, out_shape=(_sds((_PR, _D)), _sds((_PR, _D))))(h0, degp)

    prop = _sc_prop(ec, _NC)
    p1 = packed(prop(table(u0), src2, dst2, zeros))
    u1 = pl.pallas_call(_mid_body, out_shape=_sds((_PR, _D)))(p1, u0, dinv)
    p2 = packed(prop(table(u1), src2, dst2, zeros))
    u2 = pl.pallas_call(_bn_body, out_shape=_sds((_PR, _D)))(
        p2, u1, dinv, b1r, gammar, betar)
    p3 = packed(prop(table(u2), src2, dst2, zeros))
    out_t = pl.pallas_call(_fin_body, out_shape=_sds((_C, _N)))(
        p3, u2, dinv, W2, b2r)
    return out_t.T
